# Initial kernel scaffold; baseline (speedup 1.0000x reference)
import jax
import jax.numpy as jnp
from jax.experimental import pallas as pl


def kernel(tokens, table):
    raise NotImplementedError("write your pallas kernel here")


# speedup vs baseline: 4.5664x; 4.5664x over previous
import functools
import math

import jax
import jax.numpy as jnp
from jax import lax
from jax.experimental import pallas as pl
from jax.experimental.pallas import tpu as pltpu
from jax.experimental.pallas import tpu_sc as plsc

EMB_D = 32
SCALE = math.sqrt(float(EMB_D))
NC, NS, LANES = 2, 16, 16
NW = NC * NS

SUB = 128
NSUB = 8
R = SUB * NSUB


def _emb_body(idx_hbm, table_hbm, out_hbm, idx_v, rows_v, sem, *, b_per_w):
    wid = lax.axis_index("s") * NC + lax.axis_index("c")
    base = wid * b_per_w
    n_groups = b_per_w // R

    @pl.loop(0, n_groups)
    def _group(g):
        off = base + g * R
        pltpu.sync_copy(idx_hbm.at[pl.ds(off, R)], idx_v)
        descs = []
        for j in range(NSUB):
            idx_sl = idx_v.at[pl.ds(j * SUB, SUB)]
            dst = rows_v.at[pl.ds(j * SUB, SUB), :]
            descs.append(pltpu.async_copy(table_hbm.at[idx_sl], dst, sem))
        for d in descs:
            d.wait()

        @plsc.parallel_loop(0, R, unroll=8)
        def _scale(r):
            rows_v[r, pl.ds(0, LANES)] = rows_v[r, pl.ds(0, LANES)] * SCALE
            rows_v[r, pl.ds(LANES, LANES)] = (
                rows_v[r, pl.ds(LANES, LANES)] * SCALE
            )

        pltpu.sync_copy(rows_v, out_hbm.at[pl.ds(off, R), :])


def kernel(tokens, table):
    B, L = tokens.shape
    n = B * L
    assert n % (NW * R) == 0
    b_per_w = n // NW
    idx = tokens.reshape(n)

    mesh = plsc.VectorSubcoreMesh(core_axis_name="c", subcore_axis_name="s")
    emb = pl.kernel(
        functools.partial(_emb_body, b_per_w=b_per_w),
        out_type=jax.ShapeDtypeStruct((n, EMB_D), jnp.float32),
        mesh=mesh,
        scratch_types=[
            pltpu.VMEM((R,), jnp.int32),
            pltpu.VMEM((R, EMB_D), jnp.float32),
            pltpu.SemaphoreType.DMA,
        ],
        compiler_params=pltpu.CompilerParams(use_tc_tiling_on_sc=False),
    )
    out = emb(idx, table)
    return out.reshape(B, L, EMB_D)


# speedup vs baseline: 5.0031x; 1.0956x over previous
import functools
import math

import jax
import jax.numpy as jnp
from jax import lax
from jax.experimental import pallas as pl
from jax.experimental.pallas import tpu as pltpu
from jax.experimental.pallas import tpu_sc as plsc

EMB_D = 32
SCALE = math.sqrt(float(EMB_D))
NC, NS, LANES = 2, 16, 16
NW = NC * NS

SUB = 128
NSUB = 8
R = SUB * NSUB


def _emb_body(idx_hbm, table_hbm, out_hbm, idx_v, rows_v, sem_i, sem_g, sem_o,
              *, b_per_w):
    wid = lax.axis_index("s") * NC + lax.axis_index("c")
    base = wid * b_per_w
    n_groups = b_per_w // R

    def fire_gathers(b):
        for j in range(NSUB):
            idx_sl = idx_v.at[b, pl.ds(j * SUB, SUB)]
            dst = rows_v.at[b, pl.ds(j * SUB, SUB), :]
            pltpu.async_copy(table_hbm.at[idx_sl], dst, sem_g.at[b])

    def drain_gathers(b):
        pltpu.make_async_copy(
            table_hbm.at[idx_v.at[b]], rows_v.at[b], sem_g.at[b]
        ).wait()

    def out_slice(gg):
        return out_hbm.at[pl.ds(base + gg * R, R), :]

    pltpu.sync_copy(idx_hbm.at[pl.ds(base, R)], idx_v.at[0])
    fire_gathers(0)
    pltpu.async_copy(idx_hbm.at[pl.ds(base + R, R)], idx_v.at[1], sem_i.at[1])

    @pl.loop(0, n_groups, step=2)
    def _pair(g0):
        for b in (0, 1):
            gg = g0 + b
            bb = 1 - b
            drain_gathers(b)

            @pl.when(gg + 2 < n_groups)
            def _prefetch_idx():
                src = idx_hbm.at[pl.ds(base + (gg + 2) * R, R)]
                pltpu.async_copy(src, idx_v.at[b], sem_i.at[b])

            @pl.when(gg + 1 < n_groups)
            def _next_gathers():
                pltpu.make_async_copy(
                    idx_hbm.at[pl.ds(base, R)], idx_v.at[bb], sem_i.at[bb]
                ).wait()

                @pl.when(gg >= 1)
                def _reuse():
                    pltpu.make_async_copy(
                        rows_v.at[bb], out_slice(0), sem_o.at[bb]
                    ).wait()

                fire_gathers(bb)

            @plsc.parallel_loop(0, R, unroll=8)
            def _scale(r):
                rows_v[b, r, pl.ds(0, LANES)] = (
                    rows_v[b, r, pl.ds(0, LANES)] * SCALE
                )
                rows_v[b, r, pl.ds(LANES, LANES)] = (
                    rows_v[b, r, pl.ds(LANES, LANES)] * SCALE
                )

            pltpu.async_copy(rows_v.at[b], out_slice(gg), sem_o.at[b])

    for b in (0, 1):
        pltpu.make_async_copy(rows_v.at[b], out_slice(0), sem_o.at[b]).wait()


def kernel(tokens, table):
    B, L = tokens.shape
    n = B * L
    assert n % (NW * R) == 0
    b_per_w = n // NW
    idx = tokens.reshape(n)

    mesh = plsc.VectorSubcoreMesh(core_axis_name="c", subcore_axis_name="s")
    emb = pl.kernel(
        functools.partial(_emb_body, b_per_w=b_per_w),
        out_type=jax.ShapeDtypeStruct((n, EMB_D), jnp.float32),
        mesh=mesh,
        scratch_types=[
            pltpu.VMEM((2, R), jnp.int32),
            pltpu.VMEM((2, R, EMB_D), jnp.float32),
            pltpu.SemaphoreType.DMA((2,)),
            pltpu.SemaphoreType.DMA((2,)),
            pltpu.SemaphoreType.DMA((2,)),
        ],
        compiler_params=pltpu.CompilerParams(use_tc_tiling_on_sc=False),
    )
    out = emb(idx, table)
    return out.reshape(B, L, EMB_D)
